# baseline (speedup 1.0000x reference)
import functools

import jax
import jax.numpy as jnp
from jax import lax
from jax.experimental import pallas as pl
from jax.experimental.pallas import tpu as pltpu
from jax.experimental.pallas import tpu_sc as plsc

EMB_DIM = 128
NUM_EDGES = 320000
NC = 2
NS = 16
NW = NC * NS
CH = 128
RPW = 80
PAD_ROWS = NW * RPW * CH - NUM_EDGES
NB = 2
NCHUNK = NUM_EDGES // CH
LAST_W_CHUNKS = NCHUNK - (NW - 1) * RPW


@functools.cache
def _build_gather_kernel():
    @functools.partial(
        pl.kernel,
        mesh=plsc.VectorSubcoreMesh(core_axis_name="c", subcore_axis_name="s"),
        out_type=jax.ShapeDtypeStruct((NUM_EDGES, EMB_DIM), jnp.float32),
        scratch_types=(
            [pltpu.VMEM((RPW, CH), jnp.int32),
             pltpu.VMEM((NB, CH, EMB_DIM), jnp.float32),
             pltpu.VMEM_SHARED((9, EMB_DIM), jnp.float32)]
            + [pltpu.SemaphoreType.DMA] * (2 * NB)
        ),
    )
    def _gather_kernel(idx_hbm, table_hbm, out_hbm, idx_v, rows_v, table_s,
                       *sems):
        gsems, wsems = sems[:NB], sems[NB:]
        cid = lax.axis_index("c")
        sid = lax.axis_index("s")
        wid = sid * NC + cid
        row0 = wid * RPW * CH
        n_passes = jnp.where(wid == NW - 1, LAST_W_CHUNKS // NB, RPW // NB)

        @pl.when(sid == 0)
        def _():
            pltpu.sync_copy(table_hbm, table_s)

        pltpu.sync_copy(idx_hbm.at[pl.ds(wid * RPW, RPW)], idx_v)
        plsc.subcore_barrier()

        def body(g, carry):
            jbase = g * NB
            gd = []
            for b in range(NB):
                @pl.when(g > 0)
                def _():
                    pltpu.make_async_copy(
                        rows_v.at[b],
                        out_hbm.at[pl.ds(row0 + (jbase - NB + b) * CH, CH)],
                        wsems[b]).wait()
                gd.append(pltpu.async_copy(
                    table_s.at[idx_v.at[jbase + b]], rows_v.at[b], gsems[b]))
            for b in range(NB):
                gd[b].wait()
                pltpu.async_copy(
                    rows_v.at[b],
                    out_hbm.at[pl.ds(row0 + (jbase + b) * CH, CH)],
                    wsems[b])
            return carry

        lax.fori_loop(0, n_passes, body, 0)
        jlast = (n_passes - 1) * NB
        for b in range(NB):
            pltpu.make_async_copy(
                rows_v.at[b],
                out_hbm.at[pl.ds(row0 + (jlast + b) * CH, CH)],
                wsems[b]).wait()

    return _gather_kernel


def kernel(edge_attr, emb_table_0):
    idx = jnp.concatenate(
        [edge_attr.astype(jnp.int32),
         jnp.zeros((PAD_ROWS, 1), jnp.int32)], axis=0)
    idx = idx.reshape(NW * RPW, CH)
    return _build_gather_kernel()(idx, emb_table_0)


# speedup vs baseline: 1.3967x; 1.3967x over previous
import functools

import jax
import jax.numpy as jnp
from jax import lax
from jax.experimental import pallas as pl
from jax.experimental.pallas import tpu as pltpu
from jax.experimental.pallas import tpu_sc as plsc

EMB_DIM = 128
NUM_EDGES = 320000
NC = 2
NS = 16
NW = NC * NS
CH = 128
RPW = 80
PAD_ROWS = NW * RPW * CH - NUM_EDGES
NB = 4
NCHUNK = NUM_EDGES // CH
W30_CHUNKS = 48
W31_CHUNKS = 52


@functools.cache
def _build_gather_kernel():
    @functools.partial(
        pl.kernel,
        mesh=plsc.VectorSubcoreMesh(core_axis_name="c", subcore_axis_name="s"),
        out_type=jax.ShapeDtypeStruct((NUM_EDGES, EMB_DIM), jnp.float32),
        scratch_types=(
            [pltpu.VMEM((RPW, CH), jnp.int32),
             pltpu.VMEM((NB, CH, EMB_DIM), jnp.float32),
             pltpu.VMEM_SHARED((9, EMB_DIM), jnp.float32)]
            + [pltpu.SemaphoreType.DMA] * (2 * NB)
        ),
    )
    def _gather_kernel(idx_hbm, table_hbm, out_hbm, idx_v, rows_v, table_s,
                       *sems):
        gsems, wsems = sems[:NB], sems[NB:]
        cid = lax.axis_index("c")
        sid = lax.axis_index("s")
        wid = sid * NC + cid
        r0 = jnp.where(wid == NW - 1, (NW - 2) * RPW + W30_CHUNKS, wid * RPW)
        row0 = r0 * CH
        n_passes = jnp.where(
            wid == NW - 2, W30_CHUNKS // NB,
            jnp.where(wid == NW - 1, W31_CHUNKS // NB, RPW // NB))

        @pl.when(sid == 0)
        def _():
            pltpu.sync_copy(table_hbm, table_s)

        pltpu.sync_copy(idx_hbm.at[pl.ds(r0, RPW)], idx_v)
        plsc.subcore_barrier()

        def body(g, carry):
            jbase = g * NB
            gd = []
            for b in range(NB):
                @pl.when(g > 0)
                def _():
                    pltpu.make_async_copy(
                        rows_v.at[b],
                        out_hbm.at[pl.ds(row0 + (jbase - NB + b) * CH, CH)],
                        wsems[b]).wait()
                gd.append(pltpu.async_copy(
                    table_s.at[idx_v.at[jbase + b]], rows_v.at[b], gsems[b]))
            for b in range(NB):
                gd[b].wait()
                pltpu.async_copy(
                    rows_v.at[b],
                    out_hbm.at[pl.ds(row0 + (jbase + b) * CH, CH)],
                    wsems[b])
            return carry

        lax.fori_loop(0, n_passes, body, 0)
        jlast = (n_passes - 1) * NB
        for b in range(NB):
            pltpu.make_async_copy(
                rows_v.at[b],
                out_hbm.at[pl.ds(row0 + (jlast + b) * CH, CH)],
                wsems[b]).wait()

    return _gather_kernel


def kernel(edge_attr, emb_table_0):
    idx = jnp.concatenate(
        [edge_attr.astype(jnp.int32),
         jnp.zeros((PAD_ROWS, 1), jnp.int32)], axis=0)
    idx = idx.reshape(NW * RPW, CH)
    return _build_gather_kernel()(idx, emb_table_0)


# speedup vs baseline: 1.4140x; 1.0124x over previous
import functools

import jax
import jax.numpy as jnp
from jax import lax
from jax.experimental import pallas as pl
from jax.experimental.pallas import tpu as pltpu
from jax.experimental.pallas import tpu_sc as plsc

EMB_DIM = 128
NUM_EDGES = 320000
NC = 2
NS = 16
NW = NC * NS
CH = 128
RPW = 80
PAD_ROWS = NW * RPW * CH - NUM_EDGES
NB = 8
NCHUNK = NUM_EDGES // CH
W30_CHUNKS = 48
W31_CHUNKS = 52


@functools.cache
def _build_gather_kernel():
    @functools.partial(
        pl.kernel,
        mesh=plsc.VectorSubcoreMesh(core_axis_name="c", subcore_axis_name="s"),
        out_type=jax.ShapeDtypeStruct((NUM_EDGES, EMB_DIM), jnp.float32),
        scratch_types=(
            [pltpu.VMEM((RPW, CH), jnp.int32),
             pltpu.VMEM((NB, CH // 2, EMB_DIM), jnp.float32),
             pltpu.VMEM_SHARED((9, EMB_DIM), jnp.float32)]
            + [pltpu.SemaphoreType.DMA] * (2 * NB)
        ),
    )
    def _gather_kernel(idx_hbm, table_hbm, out_hbm, idx_v, rows_v, table_s,
                       *sems):
        gsems, wsems = sems[:NB], sems[NB:]
        cid = lax.axis_index("c")
        sid = lax.axis_index("s")
        wid = sid * NC + cid
        r0 = jnp.where(wid == NW - 1, (NW - 2) * RPW + W30_CHUNKS, wid * RPW)
        row0 = r0 * CH
        n_passes = jnp.where(
            wid == NW - 2, 2 * W30_CHUNKS // NB,
            jnp.where(wid == NW - 1, 2 * W31_CHUNKS // NB, 2 * RPW // NB))

        @pl.when(sid == 0)
        def _():
            pltpu.sync_copy(table_hbm, table_s)

        pltpu.sync_copy(idx_hbm.at[pl.ds(r0, RPW)], idx_v)
        plsc.subcore_barrier()

        def body(g, carry):
            jbase = g * NB
            gd = []
            H = CH // 2
            for b in range(NB):
                j = jbase + b
                @pl.when(g > 0)
                def _():
                    pltpu.make_async_copy(
                        rows_v.at[b],
                        out_hbm.at[pl.ds(row0 + (j - NB) * H, H)],
                        wsems[b]).wait()
                gd.append(pltpu.async_copy(
                    table_s.at[idx_v.at[j // 2, pl.ds((j % 2) * H, H)]],
                    rows_v.at[b], gsems[b]))
            for b in range(NB):
                j = jbase + b
                gd[b].wait()
                pltpu.async_copy(
                    rows_v.at[b],
                    out_hbm.at[pl.ds(row0 + j * H, H)],
                    wsems[b])
            return carry

        lax.fori_loop(0, n_passes, body, 0)
        jlast = (n_passes - 1) * NB
        for b in range(NB):
            pltpu.make_async_copy(
                rows_v.at[b],
                out_hbm.at[pl.ds(row0 + (jlast + b) * (CH // 2), CH // 2)],
                wsems[b]).wait()

    return _gather_kernel


def kernel(edge_attr, emb_table_0):
    idx = jnp.concatenate(
        [edge_attr.astype(jnp.int32),
         jnp.zeros((PAD_ROWS, 1), jnp.int32)], axis=0)
    idx = idx.reshape(NW * RPW, CH)
    return _build_gather_kernel()(idx, emb_table_0)
